# baseline (speedup 1.0000x reference)
import functools
import jax
import jax.numpy as jnp
from jax import lax
from jax.experimental import pallas as pl
from jax.experimental.pallas import tpu as pltpu
from jax.experimental.pallas import tpu_sc as plsc

T = 32768
D = 768
E = 64
TB = 2048
HALF = T // 2

NC = 2
NS = 16
NW = NC * NS
TPW = HALF // NW
C = 128


def _matmul_body(x_ref, sim_ref, logits_ref):
    x = x_ref[...]
    sim = sim_ref[...]
    sn = sim / jnp.clip(
        jnp.sqrt(jnp.sum(sim * sim, axis=0, keepdims=True)), 1e-12
    )
    xn = x * (1.0 / jnp.maximum(
        jnp.sqrt(jnp.sum(x * x, axis=1, keepdims=True)), 1e-12))
    logits_ref[...] = jnp.dot(xn, sn, preferred_element_type=jnp.float32)


def _tc_logits(x, sim_matrix):
    n = x.shape[0]
    return pl.pallas_call(
        _matmul_body,
        grid=(n // TB,),
        in_specs=[
            pl.BlockSpec((TB, D), lambda i: (i, 0)),
            pl.BlockSpec((D, E), lambda i: (0, 0)),
        ],
        out_specs=pl.BlockSpec((TB, E), lambda i: (i, 0)),
        out_shape=jax.ShapeDtypeStruct((n, E), jnp.float32),
        compiler_params=pltpu.CompilerParams(
            dimension_semantics=("arbitrary",),
        ),
    )(x, sim_matrix)


def _fused_body(x_ref, sim_ref, gates_ref, mask_ref, probs_ref, logits_ref):
    x = x_ref[...]
    sim = sim_ref[...]
    g = gates_ref[...]
    sn = sim / jnp.clip(
        jnp.sqrt(jnp.sum(sim * sim, axis=0, keepdims=True)), 1e-12
    )
    xn = x * (1.0 / jnp.maximum(
        jnp.sqrt(jnp.sum(x * x, axis=1, keepdims=True)), 1e-12))
    logits = jnp.dot(xn, sn, preferred_element_type=jnp.float32)
    thr = 1.0 / (1.0 + jnp.exp(-g))
    gated = jnp.maximum(logits - thr, 0.0)
    mask = (gated > 0.0).astype(jnp.float32)
    inactive = jnp.sum(mask, axis=1, keepdims=True) == 0.0
    col = jax.lax.broadcasted_iota(jnp.int32, logits.shape, 1)
    rowmax = jnp.max(logits, axis=1, keepdims=True)
    idx = jnp.where(logits == rowmax, col, jnp.int32(E))
    top1 = jnp.min(idx, axis=1, keepdims=True)
    onehot = col == top1
    mask = jnp.where(inactive & onehot, 1.0, mask)
    gm = jnp.where(mask > 0.0, gated, jnp.float32(-1e9))
    m2 = jnp.max(gm, axis=1, keepdims=True)
    ex = jnp.exp(gm - m2)
    probs = ex / jnp.sum(ex, axis=1, keepdims=True)
    mask_ref[...] = mask
    probs_ref[...] = probs
    logits_ref[...] = logits


def _tc_fused(x, sim_matrix, gates2d):
    n = x.shape[0]
    out_shapes = (
        jax.ShapeDtypeStruct((n, E), jnp.float32),
        jax.ShapeDtypeStruct((n, E), jnp.float32),
        jax.ShapeDtypeStruct((n, E), jnp.float32),
    )
    return pl.pallas_call(
        _fused_body,
        grid=(n // TB,),
        in_specs=[
            pl.BlockSpec((TB, D), lambda i: (i, 0)),
            pl.BlockSpec((D, E), lambda i: (0, 0)),
            pl.BlockSpec((1, E), lambda i: (0, 0)),
        ],
        out_specs=tuple(
            pl.BlockSpec((TB, E), lambda i: (i, 0)) for _ in range(3)),
        out_shape=out_shapes,
        compiler_params=pltpu.CompilerParams(
            dimension_semantics=("arbitrary",),
        ),
    )(x, sim_matrix, gates2d)


def _routing_body(logits_hbm, gates_hbm, mask_hbm, probs_hbm,
                  g_v, lg0, lg1, mk0, mk1, pb0, pb1,
                  si0, si1, sm0, sm1, sp0, sp1):
    wid = lax.axis_index("s") * NC + lax.axis_index("c")
    base = wid * TPW

    lg = (lg0, lg1)
    mk = (mk0, mk1)
    pb = (pb0, pb1)
    si = (si0, si1)
    sm = (sm0, sm1)
    sp = (sp0, sp1)

    pltpu.sync_copy(gates_hbm, g_v)
    thr = []
    ids = []
    for j in range(4):
        g = g_v[pl.ds(j * 16, 16)]
        thr.append(1.0 / (1.0 + jnp.exp(-g)))
        ids.append(lax.broadcasted_iota(jnp.int32, (16,), 0) + (j * 16))

    NCH = TPW // C
    h_in = [None] * NCH
    h_out = [None] * NCH

    def start_in(c):
        b = c % 2
        return pltpu.async_copy(
            logits_hbm.at[pl.ds(base + c * C, C), :], lg[b], si[b])

    h_in[0] = start_in(0)
    for c in range(NCH):
        b = c % 2
        if c + 1 < NCH:
            h_in[c + 1] = start_in(c + 1)
        h_in[c].wait()
        if c >= 2:
            hm, hp = h_out[c - 2]
            hm.wait()
            hp.wait()

        def body(t, _b=b):
            l = [lg[_b][t, pl.ds(j * 16, 16)] for j in range(4)]
            gated = [jnp.maximum(l[j] - thr[j], 0.0) for j in range(4)]
            mask = [jnp.sign(gated[j]) for j in range(4)]
            gmax = jnp.max(jnp.maximum(jnp.maximum(gated[0], gated[1]),
                                       jnp.maximum(gated[2], gated[3])))
            mx = jnp.max(jnp.maximum(jnp.maximum(l[0], l[1]),
                                     jnp.maximum(l[2], l[3])))
            idx = [jnp.where(l[j] == mx, ids[j], jnp.int32(E))
                   for j in range(4)]
            top1 = jnp.min(jnp.minimum(jnp.minimum(idx[0], idx[1]),
                                       jnp.minimum(idx[2], idx[3])))
            inactive = gmax == 0.0
            mask = [jnp.where(jnp.logical_and(inactive, ids[j] == top1),
                              1.0, mask[j]) for j in range(4)]
            gm = [jnp.where(mask[j] > 0.0, gated[j], jnp.float32(-1e9))
                  for j in range(4)]
            ex = [jnp.exp(gm[j] - gmax) for j in range(4)]
            s = jnp.sum(ex[0] + ex[1] + ex[2] + ex[3])
            invv = 1.0 / (jnp.zeros((16,), jnp.float32) + s)
            for j in range(4):
                mk[_b][t, pl.ds(j * 16, 16)] = mask[j]
                pb[_b][t, pl.ds(j * 16, 16)] = ex[j] * invv

        plsc.parallel_loop(0, C, 1, unroll=8)(body)

        tok = base + c * C
        h_out[c] = (
            pltpu.async_copy(mk[b], mask_hbm.at[pl.ds(tok, C), :], sm[b]),
            pltpu.async_copy(pb[b], probs_hbm.at[pl.ds(tok, C), :], sp[b]),
        )

    for c in (NCH - 2, NCH - 1):
        hm, hp = h_out[c]
        hm.wait()
        hp.wait()


def _sc_routing(logits, gates):
    n = logits.shape[0]
    mesh = plsc.VectorSubcoreMesh(
        core_axis_name="c", subcore_axis_name="s",
        num_cores=NC, num_subcores=NS)
    fn = functools.partial(
        pl.kernel,
        mesh=mesh,
        compiler_params=pltpu.CompilerParams(
            use_tc_tiling_on_sc=True, needs_layout_passes=False),
        out_type=(
            jax.ShapeDtypeStruct((n, E), jnp.float32),
            jax.ShapeDtypeStruct((n, E), jnp.float32),
        ),
        scratch_types=[
            pltpu.VMEM((E,), jnp.float32),
            pltpu.VMEM((C, E), jnp.float32),
            pltpu.VMEM((C, E), jnp.float32),
            pltpu.VMEM((C, E), jnp.float32),
            pltpu.VMEM((C, E), jnp.float32),
            pltpu.VMEM((C, E), jnp.float32),
            pltpu.VMEM((C, E), jnp.float32),
            pltpu.SemaphoreType.DMA,
            pltpu.SemaphoreType.DMA,
            pltpu.SemaphoreType.DMA,
            pltpu.SemaphoreType.DMA,
            pltpu.SemaphoreType.DMA,
            pltpu.SemaphoreType.DMA,
        ],
    )(_routing_body)
    return fn(logits, gates)


def kernel(x, sim_matrix, gates):
    gates2d = gates.reshape(1, E)
    xa = lax.slice_in_dim(x, 0, HALF, axis=0)
    xb = lax.slice_in_dim(x, HALF, T, axis=0)
    lga = _tc_logits(xa, sim_matrix)
    ma, pa = _sc_routing(lga, gates)
    mb, pb_, lgb = _tc_fused(xb, sim_matrix, gates2d)
    return (
        jnp.concatenate([ma, mb], axis=0),
        jnp.concatenate([pa, pb_], axis=0),
        jnp.concatenate([lga, lgb], axis=0),
    )


# speedup vs baseline: 1.5340x; 1.5340x over previous
import functools
import jax
import jax.numpy as jnp
from jax import lax
from jax.experimental import pallas as pl
from jax.experimental.pallas import tpu as pltpu
from jax.experimental.pallas import tpu_sc as plsc

T = 32768
D = 768
E = 64
TB = 4096

NC = 2
NS = 16
NW = NC * NS
TPW = T // NW
C = 128


def _matmul_body(x_ref, sim_ref, logits_ref):
    x = x_ref[...]
    sim = sim_ref[...]
    sn = sim / jnp.clip(
        jnp.sqrt(jnp.sum(sim * sim, axis=0, keepdims=True)), 1e-12
    )
    xn = x * (1.0 / jnp.maximum(
        jnp.sqrt(jnp.sum(x * x, axis=1, keepdims=True)), 1e-12))
    logits_ref[...] = jnp.dot(xn, sn, preferred_element_type=jnp.float32)


def _tc_logits(x, sim_matrix):
    return pl.pallas_call(
        _matmul_body,
        grid=(T // TB,),
        in_specs=[
            pl.BlockSpec((TB, D), lambda i: (i, 0)),
            pl.BlockSpec((D, E), lambda i: (0, 0)),
        ],
        out_specs=pl.BlockSpec((TB, E), lambda i: (i, 0)),
        out_shape=jax.ShapeDtypeStruct((T, E), jnp.float32),
        compiler_params=pltpu.CompilerParams(
            dimension_semantics=("arbitrary",),
        ),
    )(x, sim_matrix)


def _routing_body(logits_hbm, gates_hbm, mask_hbm, probs_hbm,
                  g_v, lg0, lg1, mk0, mk1, pb0, pb1,
                  si0, si1, sm0, sm1, sp0, sp1):
    wid = lax.axis_index("s") * NC + lax.axis_index("c")
    base = wid * TPW

    lg = (lg0, lg1)
    mk = (mk0, mk1)
    pb = (pb0, pb1)
    si = (si0, si1)
    sm = (sm0, sm1)
    sp = (sp0, sp1)

    pltpu.sync_copy(gates_hbm, g_v)
    thr = []
    ids = []
    for j in range(4):
        g = g_v[pl.ds(j * 16, 16)]
        thr.append(1.0 / (1.0 + jnp.exp(-g)))
        ids.append(lax.broadcasted_iota(jnp.int32, (16,), 0) + (j * 16))

    NCH = TPW // C
    h_in = [None] * NCH
    h_out = [None] * NCH

    def start_in(c):
        b = c % 2
        return pltpu.async_copy(
            logits_hbm.at[pl.ds(base + c * C, C), :], lg[b], si[b])

    h_in[0] = start_in(0)
    for c in range(NCH):
        b = c % 2
        if c + 1 < NCH:
            h_in[c + 1] = start_in(c + 1)
        h_in[c].wait()
        if c >= 2:
            hm, hp = h_out[c - 2]
            hm.wait()
            hp.wait()

        def body(t, _b=b):
            l = [lg[_b][t, pl.ds(j * 16, 16)] for j in range(4)]
            gated = [jnp.maximum(l[j] - thr[j], 0.0) for j in range(4)]
            mask = [jnp.sign(gated[j]) for j in range(4)]
            gmax = jnp.max(jnp.maximum(jnp.maximum(gated[0], gated[1]),
                                       jnp.maximum(gated[2], gated[3])))
            mx = jnp.max(jnp.maximum(jnp.maximum(l[0], l[1]),
                                     jnp.maximum(l[2], l[3])))
            idx = [jnp.where(l[j] == mx, ids[j], jnp.int32(E))
                   for j in range(4)]
            top1 = jnp.min(jnp.minimum(jnp.minimum(idx[0], idx[1]),
                                       jnp.minimum(idx[2], idx[3])))
            inactive = gmax == 0.0
            mask = [jnp.where(jnp.logical_and(inactive, ids[j] == top1),
                              1.0, mask[j]) for j in range(4)]
            gm = [jnp.where(mask[j] > 0.0, gated[j], jnp.float32(-1e9))
                  for j in range(4)]
            ex = [jnp.exp(gm[j] - gmax) for j in range(4)]
            s = jnp.sum(ex[0] + ex[1] + ex[2] + ex[3])
            invv = 1.0 / (jnp.zeros((16,), jnp.float32) + s)
            for j in range(4):
                mk[_b][t, pl.ds(j * 16, 16)] = mask[j]
                pb[_b][t, pl.ds(j * 16, 16)] = ex[j] * invv

        plsc.parallel_loop(0, C, 1, unroll=16)(body)

        tok = base + c * C
        h_out[c] = (
            pltpu.async_copy(mk[b], mask_hbm.at[pl.ds(tok, C), :], sm[b]),
            pltpu.async_copy(pb[b], probs_hbm.at[pl.ds(tok, C), :], sp[b]),
        )

    for c in (NCH - 2, NCH - 1):
        hm, hp = h_out[c]
        hm.wait()
        hp.wait()


def _sc_routing(logits, gates):
    mesh = plsc.VectorSubcoreMesh(
        core_axis_name="c", subcore_axis_name="s",
        num_cores=NC, num_subcores=NS)
    fn = functools.partial(
        pl.kernel,
        mesh=mesh,
        compiler_params=pltpu.CompilerParams(
            use_tc_tiling_on_sc=True, needs_layout_passes=False),
        out_type=(
            jax.ShapeDtypeStruct((T, E), jnp.float32),
            jax.ShapeDtypeStruct((T, E), jnp.float32),
        ),
        scratch_types=[
            pltpu.VMEM((E,), jnp.float32),
            pltpu.VMEM((C, E), jnp.float32),
            pltpu.VMEM((C, E), jnp.float32),
            pltpu.VMEM((C, E), jnp.float32),
            pltpu.VMEM((C, E), jnp.float32),
            pltpu.VMEM((C, E), jnp.float32),
            pltpu.VMEM((C, E), jnp.float32),
            pltpu.SemaphoreType.DMA,
            pltpu.SemaphoreType.DMA,
            pltpu.SemaphoreType.DMA,
            pltpu.SemaphoreType.DMA,
            pltpu.SemaphoreType.DMA,
            pltpu.SemaphoreType.DMA,
        ],
    )(_routing_body)
    return fn(logits, gates)


def kernel(x, sim_matrix, gates):
    logits = _tc_logits(x, sim_matrix)
    mask, probs = _sc_routing(logits, gates)
    return (mask, probs, logits)
